# initial kernel scaffold (unmeasured)
import functools

import jax
import jax.numpy as jnp
from jax import lax
from jax.experimental import pallas as pl
from jax.experimental.pallas import tpu as pltpu

N_DEV = 4


def _mm_body(x_ref, w_ref, o_ref, *, relu):
    acc = jnp.dot(x_ref[...], w_ref[...], preferred_element_type=jnp.float32)
    if relu:
        acc = jnp.maximum(acc, 0.0)
    o_ref[...] = acc


def _matmul(x, w, bm, bn, relu=False):
    m, k = x.shape
    _, n = w.shape
    return pl.pallas_call(
        functools.partial(_mm_body, relu=relu),
        grid=(m // bm, n // bn),
        in_specs=[
            pl.BlockSpec((bm, k), lambda i, j: (i, 0)),
            pl.BlockSpec((k, bn), lambda i, j: (0, j)),
        ],
        out_specs=pl.BlockSpec((bm, bn), lambda i, j: (i, j)),
        out_shape=jax.ShapeDtypeStruct((m, n), jnp.float32),
        compiler_params=pltpu.CompilerParams(
            dimension_semantics=("parallel", "parallel"),
        ),
    )(x, w)


def _allreduce_body(p_ref, out_ref, comm_ref,
                    rs_send_sems, rs_recv_sems, ag_send_sems, ag_recv_sems):
    my = lax.axis_index("i")
    left = lax.rem(my + N_DEV - 1, N_DEV)
    right = lax.rem(my + 1, N_DEV)
    m, _ = out_ref.shape
    c = m // N_DEV

    barrier_sem = pltpu.get_barrier_semaphore()
    for nbr in (left, right):
        pl.semaphore_signal(
            barrier_sem, inc=1,
            device_id=(nbr,), device_id_type=pl.DeviceIdType.MESH,
        )
    pl.semaphore_wait(barrier_sem, 2)

    out_ref[...] = p_ref[...]

    for s in range(N_DEV - 1):
        send_c = lax.rem(my - s + 2 * N_DEV, N_DEV)
        rdma = pltpu.make_async_remote_copy(
            src_ref=out_ref.at[pl.ds(send_c * c, c), :],
            dst_ref=comm_ref.at[s],
            send_sem=rs_send_sems.at[s],
            recv_sem=rs_recv_sems.at[s],
            device_id=(right,),
            device_id_type=pl.DeviceIdType.MESH,
        )
        rdma.start()
        rdma.wait()
        recv_c = lax.rem(my - s - 1 + 2 * N_DEV, N_DEV)
        sl = pl.ds(recv_c * c, c)
        out_ref[sl, :] = out_ref[sl, :] + comm_ref[s]

    for s in range(N_DEV - 1):
        send_c = lax.rem(my + 1 - s + 2 * N_DEV, N_DEV)
        rdma = pltpu.make_async_remote_copy(
            src_ref=out_ref.at[pl.ds(send_c * c, c), :],
            dst_ref=out_ref.at[pl.ds(send_c * c, c), :],
            send_sem=ag_send_sems.at[s],
            recv_sem=ag_recv_sems.at[s],
            device_id=(right,),
            device_id_type=pl.DeviceIdType.MESH,
        )
        rdma.start()
        rdma.wait()


def _allreduce(p):
    m, n = p.shape
    return pl.pallas_call(
        _allreduce_body,
        out_shape=jax.ShapeDtypeStruct((m, n), jnp.float32),
        in_specs=[pl.BlockSpec(memory_space=pltpu.VMEM)],
        out_specs=pl.BlockSpec(memory_space=pltpu.VMEM),
        scratch_shapes=[
            pltpu.VMEM((N_DEV - 1, m // N_DEV, n), jnp.float32),
            pltpu.SemaphoreType.DMA((N_DEV - 1,)),
            pltpu.SemaphoreType.DMA((N_DEV - 1,)),
            pltpu.SemaphoreType.DMA((N_DEV - 1,)),
            pltpu.SemaphoreType.DMA((N_DEV - 1,)),
        ],
        compiler_params=pltpu.CompilerParams(collective_id=0),
    )(p)


def kernel(x, W1, W2):
    h = _matmul(x, W1, bm=512, bn=1024, relu=True)
    p = _matmul(h, W2, bm=512, bn=512)
    return _allreduce(p)


# baseline (device time: 439883 ns/iter reference)
import functools

import jax
import jax.numpy as jnp
from jax import lax
from jax.experimental import pallas as pl
from jax.experimental.pallas import tpu as pltpu

N_DEV = 4


def _mm_body(x_ref, w_ref, o_ref, *, relu):
    acc = jnp.dot(x_ref[...], w_ref[...], preferred_element_type=jnp.float32)
    if relu:
        acc = jnp.maximum(acc, 0.0)
    o_ref[...] = acc


def _matmul(x, w, bm, bn, relu=False):
    m, k = x.shape
    _, n = w.shape
    return pl.pallas_call(
        functools.partial(_mm_body, relu=relu),
        grid=(m // bm, n // bn),
        in_specs=[
            pl.BlockSpec((bm, k), lambda i, j: (i, 0)),
            pl.BlockSpec((k, bn), lambda i, j: (0, j)),
        ],
        out_specs=pl.BlockSpec((bm, bn), lambda i, j: (i, j)),
        out_shape=jax.ShapeDtypeStruct((m, n), jnp.float32),
        compiler_params=pltpu.CompilerParams(
            dimension_semantics=("parallel", "parallel"),
            vmem_limit_bytes=100 * 1024 * 1024,
        ),
    )(x, w)


def _allreduce_body(p_ref, out_ref, comm_ref,
                    rs_send_sems, rs_recv_sems, ag_send_sems, ag_recv_sems):
    my = lax.axis_index("i")
    left = lax.rem(my + N_DEV - 1, N_DEV)
    right = lax.rem(my + 1, N_DEV)
    m, _ = out_ref.shape
    c = m // N_DEV

    barrier_sem = pltpu.get_barrier_semaphore()
    for nbr in (left, right):
        pl.semaphore_signal(
            barrier_sem, inc=1,
            device_id=(nbr,), device_id_type=pl.DeviceIdType.MESH,
        )
    pl.semaphore_wait(barrier_sem, 2)

    out_ref[...] = p_ref[...]

    for s in range(N_DEV - 1):
        send_c = lax.rem(my - s + 2 * N_DEV, N_DEV)
        rdma = pltpu.make_async_remote_copy(
            src_ref=out_ref.at[pl.ds(send_c * c, c), :],
            dst_ref=comm_ref.at[s],
            send_sem=rs_send_sems.at[s],
            recv_sem=rs_recv_sems.at[s],
            device_id=(right,),
            device_id_type=pl.DeviceIdType.MESH,
        )
        rdma.start()
        rdma.wait()
        recv_c = lax.rem(my - s - 1 + 2 * N_DEV, N_DEV)
        sl = pl.ds(recv_c * c, c)
        out_ref[sl, :] = out_ref[sl, :] + comm_ref[s]

    for s in range(N_DEV - 1):
        send_c = lax.rem(my + 1 - s + 2 * N_DEV, N_DEV)
        rdma = pltpu.make_async_remote_copy(
            src_ref=out_ref.at[pl.ds(send_c * c, c), :],
            dst_ref=out_ref.at[pl.ds(send_c * c, c), :],
            send_sem=ag_send_sems.at[s],
            recv_sem=ag_recv_sems.at[s],
            device_id=(right,),
            device_id_type=pl.DeviceIdType.MESH,
        )
        rdma.start()
        rdma.wait()


def _allreduce(p):
    m, n = p.shape
    return pl.pallas_call(
        _allreduce_body,
        out_shape=jax.ShapeDtypeStruct((m, n), jnp.float32),
        in_specs=[pl.BlockSpec(memory_space=pltpu.VMEM)],
        out_specs=pl.BlockSpec(memory_space=pltpu.VMEM),
        scratch_shapes=[
            pltpu.VMEM((N_DEV - 1, m // N_DEV, n), jnp.float32),
            pltpu.SemaphoreType.DMA((N_DEV - 1,)),
            pltpu.SemaphoreType.DMA((N_DEV - 1,)),
            pltpu.SemaphoreType.DMA((N_DEV - 1,)),
            pltpu.SemaphoreType.DMA((N_DEV - 1,)),
        ],
        compiler_params=pltpu.CompilerParams(
            collective_id=0,
            vmem_limit_bytes=100 * 1024 * 1024,
        ),
    )(p)


def kernel(x, W1, W2):
    h = _matmul(x, W1, bm=512, bn=1024, relu=True)
    p = _matmul(h, W2, bm=512, bn=512)
    return _allreduce(p)


# device time: 305207 ns/iter; 1.4413x vs baseline; 1.4413x over previous
import functools

import jax
import jax.numpy as jnp
from jax import lax
from jax.experimental import pallas as pl
from jax.experimental.pallas import tpu as pltpu

N_DEV = 4


def _mm_body(x_ref, w_ref, o_ref, *, relu):
    acc = jnp.dot(x_ref[...], w_ref[...], preferred_element_type=jnp.float32)
    if relu:
        acc = jnp.maximum(acc, 0.0)
    o_ref[...] = acc


def _matmul(x, w, bm, bn, relu=False):
    m, k = x.shape
    _, n = w.shape
    return pl.pallas_call(
        functools.partial(_mm_body, relu=relu),
        grid=(m // bm, n // bn),
        in_specs=[
            pl.BlockSpec((bm, k), lambda i, j: (i, 0)),
            pl.BlockSpec((k, bn), lambda i, j: (0, j)),
        ],
        out_specs=pl.BlockSpec((bm, bn), lambda i, j: (i, j)),
        out_shape=jax.ShapeDtypeStruct((m, n), jnp.float32),
        compiler_params=pltpu.CompilerParams(
            dimension_semantics=("parallel", "parallel"),
            vmem_limit_bytes=100 * 1024 * 1024,
        ),
    )(x, w)


def _allreduce_body(p_ref, out_ref, comm_r, comm_l,
                    rs_send_r, rs_recv_r, ag_send_r, ag_recv_r,
                    rs_send_l, rs_recv_l, ag_send_l, ag_recv_l):
    my = lax.axis_index("i")
    left = lax.rem(my + N_DEV - 1, N_DEV)
    right = lax.rem(my + 1, N_DEV)
    m, n = out_ref.shape
    c = m // N_DEV
    hn = n // 2

    barrier_sem = pltpu.get_barrier_semaphore()
    for nbr in (left, right):
        pl.semaphore_signal(
            barrier_sem, inc=1,
            device_id=(nbr,), device_id_type=pl.DeviceIdType.MESH,
        )
    pl.semaphore_wait(barrier_sem, 2)

    out_ref[...] = p_ref[...]

    def ring_copy(src_rows, dst_ref, send_sem, recv_sem, dst_dev, col0):
        return pltpu.make_async_remote_copy(
            src_ref=out_ref.at[pl.ds(src_rows * c, c), pl.ds(col0, hn)],
            dst_ref=dst_ref,
            send_sem=send_sem,
            recv_sem=recv_sem,
            device_id=(dst_dev,),
            device_id_type=pl.DeviceIdType.MESH,
        )

    for s in range(N_DEV - 1):
        r = ring_copy(lax.rem(my - s + 2 * N_DEV, N_DEV), comm_r.at[s],
                      rs_send_r.at[s], rs_recv_r.at[s], right, 0)
        l = ring_copy(lax.rem(my + s, N_DEV), comm_l.at[s],
                      rs_send_l.at[s], rs_recv_l.at[s], left, hn)
        r.start()
        l.start()
        r.wait()
        rc = lax.rem(my - s - 1 + 2 * N_DEV, N_DEV)
        out_ref[pl.ds(rc * c, c), pl.ds(0, hn)] = (
            out_ref[pl.ds(rc * c, c), pl.ds(0, hn)] + comm_r[s]
        )
        l.wait()
        lc = lax.rem(my + s + 1, N_DEV)
        out_ref[pl.ds(lc * c, c), pl.ds(hn, hn)] = (
            out_ref[pl.ds(lc * c, c), pl.ds(hn, hn)] + comm_l[s]
        )

    for s in range(N_DEV - 1):
        sc_r = lax.rem(my + 1 - s + 2 * N_DEV, N_DEV)
        r = ring_copy(sc_r,
                      out_ref.at[pl.ds(sc_r * c, c), pl.ds(0, hn)],
                      ag_send_r.at[s], ag_recv_r.at[s], right, 0)
        sc_l = lax.rem(my - 1 + s + 2 * N_DEV, N_DEV)
        l = ring_copy(sc_l,
                      out_ref.at[pl.ds(sc_l * c, c), pl.ds(hn, hn)],
                      ag_send_l.at[s], ag_recv_l.at[s], left, hn)
        r.start()
        l.start()
        r.wait()
        l.wait()


def _allreduce(p):
    m, n = p.shape
    dma3 = pltpu.SemaphoreType.DMA((N_DEV - 1,))
    return pl.pallas_call(
        _allreduce_body,
        out_shape=jax.ShapeDtypeStruct((m, n), jnp.float32),
        in_specs=[pl.BlockSpec(memory_space=pltpu.VMEM)],
        out_specs=pl.BlockSpec(memory_space=pltpu.VMEM),
        scratch_shapes=[
            pltpu.VMEM((N_DEV - 1, m // N_DEV, n // 2), jnp.float32),
            pltpu.VMEM((N_DEV - 1, m // N_DEV, n // 2), jnp.float32),
        ] + [dma3] * 8,
        compiler_params=pltpu.CompilerParams(
            collective_id=0,
            vmem_limit_bytes=100 * 1024 * 1024,
        ),
    )(p)


def kernel(x, W1, W2):
    h = _matmul(x, W1, bm=512, bn=1024, relu=True)
    p = _matmul(h, W2, bm=512, bn=512)
    return _allreduce(p)


# device time: 273337 ns/iter; 1.6093x vs baseline; 1.1166x over previous
import functools

import jax
import jax.numpy as jnp
from jax import lax
from jax.experimental import pallas as pl
from jax.experimental.pallas import tpu as pltpu

N_DEV = 4


def _mm_body(x_ref, w_ref, o_ref, *, relu):
    acc = jnp.dot(x_ref[...], w_ref[...], preferred_element_type=jnp.float32)
    if relu:
        acc = jnp.maximum(acc, 0.0)
    o_ref[...] = acc


def _matmul(x, w, bn, relu=False):
    m, k = x.shape
    _, n = w.shape
    return pl.pallas_call(
        functools.partial(_mm_body, relu=relu),
        grid=(n // bn,),
        in_specs=[
            pl.BlockSpec((m, k), lambda j: (0, 0)),
            pl.BlockSpec((k, bn), lambda j: (0, j)),
        ],
        out_specs=pl.BlockSpec((m, bn), lambda j: (0, j)),
        out_shape=jax.ShapeDtypeStruct((m, n), jnp.float32),
        compiler_params=pltpu.CompilerParams(
            dimension_semantics=("arbitrary",),
            vmem_limit_bytes=100 * 1024 * 1024,
        ),
    )(x, w)


def _allreduce_body(p_ref, out_ref, comm_r, comm_l,
                    rs_send_r, rs_recv_r, ag_send_r, ag_recv_r,
                    rs_send_l, rs_recv_l, ag_send_l, ag_recv_l):
    my = lax.axis_index("i")
    left = lax.rem(my + N_DEV - 1, N_DEV)
    right = lax.rem(my + 1, N_DEV)
    m, n = out_ref.shape
    c = m // N_DEV
    hn = n // 2

    barrier_sem = pltpu.get_barrier_semaphore()
    for nbr in (left, right):
        pl.semaphore_signal(
            barrier_sem, inc=1,
            device_id=(nbr,), device_id_type=pl.DeviceIdType.MESH,
        )
    pl.semaphore_wait(barrier_sem, 2)

    def ring_copy(src_ref, src_rows, dst_ref, send_sem, recv_sem, dst_dev,
                  col0):
        return pltpu.make_async_remote_copy(
            src_ref=src_ref.at[pl.ds(src_rows * c, c), pl.ds(col0, hn)],
            dst_ref=dst_ref,
            send_sem=send_sem,
            recv_sem=recv_sem,
            device_id=(dst_dev,),
            device_id_type=pl.DeviceIdType.MESH,
        )

    for s in range(N_DEV - 1):
        src = p_ref if s == 0 else out_ref
        r = ring_copy(src, lax.rem(my - s + 2 * N_DEV, N_DEV), comm_r.at[s],
                      rs_send_r.at[s], rs_recv_r.at[s], right, 0)
        l = ring_copy(src, lax.rem(my + s, N_DEV), comm_l.at[s],
                      rs_send_l.at[s], rs_recv_l.at[s], left, hn)
        r.start()
        l.start()
        if s == 0:
            out_ref[...] = p_ref[...]
        r.wait()
        rc = lax.rem(my - s - 1 + 2 * N_DEV, N_DEV)
        out_ref[pl.ds(rc * c, c), pl.ds(0, hn)] = (
            out_ref[pl.ds(rc * c, c), pl.ds(0, hn)] + comm_r[s]
        )
        l.wait()
        lc = lax.rem(my + s + 1, N_DEV)
        out_ref[pl.ds(lc * c, c), pl.ds(hn, hn)] = (
            out_ref[pl.ds(lc * c, c), pl.ds(hn, hn)] + comm_l[s]
        )

    for s in range(N_DEV - 1):
        sc_r = lax.rem(my + 1 - s + 2 * N_DEV, N_DEV)
        r = ring_copy(out_ref, sc_r,
                      out_ref.at[pl.ds(sc_r * c, c), pl.ds(0, hn)],
                      ag_send_r.at[s], ag_recv_r.at[s], right, 0)
        sc_l = lax.rem(my - 1 + s + 2 * N_DEV, N_DEV)
        l = ring_copy(out_ref, sc_l,
                      out_ref.at[pl.ds(sc_l * c, c), pl.ds(hn, hn)],
                      ag_send_l.at[s], ag_recv_l.at[s], left, hn)
        r.start()
        l.start()
        r.wait()
        l.wait()


def _allreduce(p):
    m, n = p.shape
    dma3 = pltpu.SemaphoreType.DMA((N_DEV - 1,))
    return pl.pallas_call(
        _allreduce_body,
        out_shape=jax.ShapeDtypeStruct((m, n), jnp.float32),
        in_specs=[pl.BlockSpec(memory_space=pltpu.VMEM)],
        out_specs=pl.BlockSpec(memory_space=pltpu.VMEM),
        scratch_shapes=[
            pltpu.VMEM((N_DEV - 1, m // N_DEV, n // 2), jnp.float32),
            pltpu.VMEM((N_DEV - 1, m // N_DEV, n // 2), jnp.float32),
        ] + [dma3] * 8,
        compiler_params=pltpu.CompilerParams(
            collective_id=0,
            vmem_limit_bytes=100 * 1024 * 1024,
        ),
    )(p)


def kernel(x, W1, W2):
    h = _matmul(x, W1, bn=1024, relu=True)
    p = _matmul(h, W2, bn=512)
    return _allreduce(p)
